# Initial kernel scaffold; baseline (speedup 1.0000x reference)
import jax
import jax.numpy as jnp
from jax.experimental import pallas as pl


def kernel(ind_2, prop, inter):
    raise NotImplementedError("write your pallas kernel here")


# speedup vs baseline: 86.3199x; 86.3199x over previous
import jax
import jax.numpy as jnp
from jax import lax
from jax.experimental import pallas as pl
from jax.experimental.pallas import tpu as pltpu
from jax.experimental.pallas import tpu_sc as plsc

N_NODES = 10000
N_EDGES = 320000
D_FEAT = 128

NCORE = 2
NSUB = 16
CPC = D_FEAT // NCORE
EPS = N_EDGES // NSUB
CE = 200
NWIN = EPS // CE
WELE = CE * CPC
ACC = N_NODES * CPC
ZSLICE = ACC // NSUB
ZCHUNK = 8000
NQ = CPC // 16


def _sc_scatter_kernel(ind2_hbm, inter_hbm, out_hbm, idxbuf, valbuf, fidx,
                       fval, zbuf, acc):
    cid = lax.axis_index("c")
    sid = lax.axis_index("s")

    zvec = jnp.zeros((16,), jnp.float32)

    def _zero(i, _):
        zbuf[pl.ds(i * 16, 16)] = zvec
        return 0

    lax.fori_loop(0, ZCHUNK // 16, _zero, 0)
    for t in range(ZSLICE // ZCHUNK):
        pltpu.sync_copy(zbuf, acc.at[pl.ds(sid * ZSLICE + t * ZCHUNK, ZCHUNK)])
    plsc.subcore_barrier()

    col0 = lax.iota(jnp.int32, 16)
    cols = [col0 + (16 * q) for q in range(NQ)]
    cbase = cid * CPC

    def _window(w, _):
        e0 = sid * EPS + w * CE
        pltpu.sync_copy(ind2_hbm.at[pl.ds(e0, CE), pl.ds(cbase, CPC)], idxbuf)
        pltpu.sync_copy(inter_hbm.at[pl.ds(e0, CE), pl.ds(cbase, CPC)], valbuf)

        def _repack(r, _):
            for q in range(NQ):
                v = idxbuf[r, pl.ds(16 * q, 16)]
                fidx[pl.ds(r * CPC + 16 * q, 16)] = v * CPC + cols[q]
                fval[pl.ds(r * CPC + 16 * q, 16)] = valbuf[r, pl.ds(16 * q, 16)]
            return 0

        lax.fori_loop(0, CE, _repack, 0)
        pltpu.sync_copy(fval, acc.at[fidx], add=True)
        return 0

    lax.fori_loop(0, NWIN, _window, 0)
    plsc.subcore_barrier()

    pltpu.sync_copy(acc.at[pl.ds(sid * ZSLICE, ZSLICE)],
                    out_hbm.at[cid, pl.ds(sid * ZSLICE, ZSLICE)])


@jax.jit
def _run(ind2_flat, inter):
    mesh = plsc.VectorSubcoreMesh(core_axis_name="c", subcore_axis_name="s")
    scatter = pl.kernel(
        _sc_scatter_kernel,
        mesh=mesh,
        out_type=jax.ShapeDtypeStruct((NCORE, ACC), jnp.float32),
        scratch_types=[
            pltpu.VMEM((CE, CPC), jnp.int32),
            pltpu.VMEM((CE, CPC), jnp.float32),
            pltpu.VMEM((WELE,), jnp.int32),
            pltpu.VMEM((WELE,), jnp.float32),
            pltpu.VMEM((ZCHUNK,), jnp.float32),
            pltpu.VMEM_SHARED((ACC,), jnp.float32),
        ],
        compiler_params=pltpu.CompilerParams(use_tc_tiling_on_sc=False),
    )
    parts = scatter(ind2_flat, inter)
    return jnp.concatenate(
        [parts[0].reshape(N_NODES, CPC), parts[1].reshape(N_NODES, CPC)],
        axis=1,
    )


def kernel(ind_2, prop, inter):
    ind2_flat = ind_2.reshape(N_EDGES, 2 * D_FEAT)
    return _run(ind2_flat, inter)


# speedup vs baseline: 147.1105x; 1.7042x over previous
import jax
import jax.numpy as jnp
from jax import lax
from jax.experimental import pallas as pl
from jax.experimental.pallas import tpu as pltpu
from jax.experimental.pallas import tpu_sc as plsc

N_NODES = 10000
N_EDGES = 320000
D_FEAT = 128

NCORE = 2
NSUB = 16
CPC = D_FEAT // NCORE
EPS = N_EDGES // NSUB
CE = 100
NWIN = EPS // CE
NPAIR = NWIN // 2
WELE = CE * CPC
ACC = N_NODES * CPC
ZSLICE = ACC // NSUB
ZCHUNK = 8000
NQ = CPC // 16


def _sc_scatter_kernel(ind2_hbm, inter_hbm, out_hbm,
                       idx0, idx1, val0, val1, fidx0, fidx1, fval0, fval1,
                       zbuf, acc, si0, si1, sv0, sv1, ss0, ss1):
    cid = lax.axis_index("c")
    sid = lax.axis_index("s")
    idxb = (idx0, idx1)
    valb = (val0, val1)
    fidxb = (fidx0, fidx1)
    fvalb = (fval0, fval1)
    sidx = (si0, si1)
    sval = (sv0, sv1)
    ssc = (ss0, ss1)

    zvec = jnp.zeros((16,), jnp.float32)

    def _zero(i, _):
        zbuf[pl.ds(i * 16, 16)] = zvec
        return 0

    lax.fori_loop(0, ZCHUNK // 16, _zero, 0)
    for t in range(ZSLICE // ZCHUNK):
        pltpu.sync_copy(zbuf, acc.at[pl.ds(sid * ZSLICE + t * ZCHUNK, ZCHUNK)])
    plsc.subcore_barrier()

    col0 = lax.iota(jnp.int32, 16)
    cols = [col0 + (16 * q) for q in range(NQ)]
    cbase = cid * CPC

    def _issue_in(w, b):
        e0 = sid * EPS + w * CE
        pltpu.async_copy(ind2_hbm.at[pl.ds(e0, CE), pl.ds(cbase, CPC)],
                         idxb[b], sidx[b])
        pltpu.async_copy(inter_hbm.at[pl.ds(e0, CE), pl.ds(cbase, CPC)],
                         valb[b], sval[b])

    def _wait_in(w, b):
        e0 = sid * EPS + w * CE
        pltpu.make_async_copy(ind2_hbm.at[pl.ds(e0, CE), pl.ds(cbase, CPC)],
                              idxb[b], sidx[b]).wait()
        pltpu.make_async_copy(inter_hbm.at[pl.ds(e0, CE), pl.ds(cbase, CPC)],
                              valb[b], sval[b]).wait()

    def _wait_scatter(b):
        pltpu.make_async_copy(fvalb[b], acc.at[fidxb[b]], ssc[b]).wait()

    def _half(k, w, b):
        _wait_in(w, b)

        @pl.when(k > 0)
        def _():
            _wait_scatter(b)

        def _repack(r, _):
            for q in range(NQ):
                src = pl.ds(16 * q, 16)
                dst = pl.ds(r * CPC + 16 * q, 16)
                fidxb[b][dst] = idxb[b][r, src] * CPC + cols[q]
                fvalb[b][dst] = valb[b][r, src]
            return 0

        lax.fori_loop(0, CE, _repack, 0)
        pltpu.async_copy(fvalb[b], acc.at[fidxb[b]], ssc[b], add=True)

    def _pair(k, _):
        w0 = 2 * k
        _issue_in(w0 + 1, 1)
        _half(k, w0, 0)

        @pl.when(k + 1 < NPAIR)
        def _():
            _issue_in(w0 + 2, 0)

        _half(k, w0 + 1, 1)
        return 0

    _issue_in(0, 0)
    lax.fori_loop(0, NPAIR, _pair, 0)
    _wait_scatter(0)
    _wait_scatter(1)
    plsc.subcore_barrier()

    pltpu.sync_copy(acc.at[pl.ds(sid * ZSLICE, ZSLICE)],
                    out_hbm.at[cid, pl.ds(sid * ZSLICE, ZSLICE)])


@jax.jit
def _run(ind2_flat, inter):
    mesh = plsc.VectorSubcoreMesh(core_axis_name="c", subcore_axis_name="s")
    scatter = pl.kernel(
        _sc_scatter_kernel,
        mesh=mesh,
        out_type=jax.ShapeDtypeStruct((NCORE, ACC), jnp.float32),
        scratch_types=[
            pltpu.VMEM((CE, CPC), jnp.int32),
            pltpu.VMEM((CE, CPC), jnp.int32),
            pltpu.VMEM((CE, CPC), jnp.float32),
            pltpu.VMEM((CE, CPC), jnp.float32),
            pltpu.VMEM((WELE,), jnp.int32),
            pltpu.VMEM((WELE,), jnp.int32),
            pltpu.VMEM((WELE,), jnp.float32),
            pltpu.VMEM((WELE,), jnp.float32),
            pltpu.VMEM((ZCHUNK,), jnp.float32),
            pltpu.VMEM_SHARED((ACC,), jnp.float32),
            pltpu.SemaphoreType.DMA,
            pltpu.SemaphoreType.DMA,
            pltpu.SemaphoreType.DMA,
            pltpu.SemaphoreType.DMA,
            pltpu.SemaphoreType.DMA,
            pltpu.SemaphoreType.DMA,
        ],
        compiler_params=pltpu.CompilerParams(use_tc_tiling_on_sc=False),
    )
    parts = scatter(ind2_flat, inter)
    return jnp.concatenate(
        [parts[0].reshape(N_NODES, CPC), parts[1].reshape(N_NODES, CPC)],
        axis=1,
    )


def kernel(ind_2, prop, inter):
    ind2_flat = ind_2.reshape(N_EDGES, 2 * D_FEAT)
    return _run(ind2_flat, inter)


# speedup vs baseline: 147.1390x; 1.0002x over previous
import jax
import jax.numpy as jnp
from jax import lax
from jax.experimental import pallas as pl
from jax.experimental.pallas import tpu as pltpu
from jax.experimental.pallas import tpu_sc as plsc

N_NODES = 10000
N_EDGES = 320000
D_FEAT = 128

NCORE = 2
NSUB = 16
CPC = D_FEAT // NCORE
EPS = N_EDGES // NSUB
CE = 100
NWIN = EPS // CE
NPAIR = NWIN // 2
WELE = CE * CPC
ACC = N_NODES * CPC
ZSLICE = ACC // NSUB
ZCHUNK = 8000
NQ = CPC // 16


def _sc_scatter_kernel(ind2_hbm, inter_hbm, out_hbm,
                       idx0, idx1, val0, val1, fidx0, fidx1, fval0, fval1,
                       zbuf, acc, si0, si1, sv0, sv1, ss0, ss1, ssa0, ssa1,
                       zsem):
    cid = lax.axis_index("c")
    sid = lax.axis_index("s")
    idxb = (idx0, idx1)
    valb = (val0, val1)
    fidxb = (fidx0, fidx1)
    fvalb = (fval0, fval1)
    sidx = (si0, si1)
    sval = (sv0, sv1)
    ssc = (ss0, ss1)
    ssc2 = (ssa0, ssa1)

    zvec = jnp.zeros((16,), jnp.float32)

    def _zero(i, _):
        zbuf[pl.ds(i * 16, 16)] = zvec
        return 0

    lax.fori_loop(0, ZCHUNK // 16, _zero, 0)
    for t in range(ZSLICE // ZCHUNK):
        pltpu.async_copy(zbuf, acc.at[pl.ds(sid * ZSLICE + t * ZCHUNK, ZCHUNK)],
                         zsem)
    for t in range(ZSLICE // ZCHUNK):
        pltpu.make_async_copy(
            zbuf, acc.at[pl.ds(sid * ZSLICE + t * ZCHUNK, ZCHUNK)], zsem
        ).wait()
    plsc.subcore_barrier()

    col0 = lax.iota(jnp.int32, 16)
    cols = [col0 + (16 * q) for q in range(NQ)]
    cbase = cid * CPC

    def _issue_in(w, b):
        e0 = sid * EPS + w * CE
        pltpu.async_copy(ind2_hbm.at[pl.ds(e0, CE), pl.ds(cbase, CPC)],
                         idxb[b], sidx[b])
        pltpu.async_copy(inter_hbm.at[pl.ds(e0, CE), pl.ds(cbase, CPC)],
                         valb[b], sval[b])

    def _wait_in(w, b):
        e0 = sid * EPS + w * CE
        pltpu.make_async_copy(ind2_hbm.at[pl.ds(e0, CE), pl.ds(cbase, CPC)],
                              idxb[b], sidx[b]).wait()
        pltpu.make_async_copy(inter_hbm.at[pl.ds(e0, CE), pl.ds(cbase, CPC)],
                              valb[b], sval[b]).wait()

    H = WELE // 2

    def _wait_scatter(b):
        pltpu.make_async_copy(fvalb[b].at[pl.ds(0, H)],
                              acc.at[fidxb[b].at[pl.ds(0, H)]], ssc[b]).wait()
        pltpu.make_async_copy(fvalb[b].at[pl.ds(H, H)],
                              acc.at[fidxb[b].at[pl.ds(H, H)]], ssc2[b]).wait()

    def _half(k, w, b):
        _wait_in(w, b)

        @pl.when(k > 0)
        def _():
            _wait_scatter(b)

        def _repack(r, _):
            for q in range(NQ):
                src = pl.ds(16 * q, 16)
                dst = pl.ds(r * CPC + 16 * q, 16)
                fidxb[b][dst] = idxb[b][r, src] * CPC + cols[q]
                fvalb[b][dst] = valb[b][r, src]
            return 0

        lax.fori_loop(0, CE, _repack, 0)
        pltpu.async_copy(fvalb[b].at[pl.ds(0, H)],
                         acc.at[fidxb[b].at[pl.ds(0, H)]], ssc[b], add=True)
        pltpu.async_copy(fvalb[b].at[pl.ds(H, H)],
                         acc.at[fidxb[b].at[pl.ds(H, H)]], ssc2[b], add=True)

    def _pair(k, _):
        w0 = 2 * k
        _issue_in(w0 + 1, 1)
        _half(k, w0, 0)

        @pl.when(k + 1 < NPAIR)
        def _():
            _issue_in(w0 + 2, 0)

        _half(k, w0 + 1, 1)
        return 0

    _issue_in(0, 0)
    lax.fori_loop(0, NPAIR, _pair, 0)
    _wait_scatter(0)
    _wait_scatter(1)
    plsc.subcore_barrier()

    pltpu.sync_copy(acc.at[pl.ds(sid * ZSLICE, ZSLICE)],
                    out_hbm.at[cid, pl.ds(sid * ZSLICE, ZSLICE)])


@jax.jit
def _run(ind2_flat, inter):
    mesh = plsc.VectorSubcoreMesh(core_axis_name="c", subcore_axis_name="s")
    scatter = pl.kernel(
        _sc_scatter_kernel,
        mesh=mesh,
        out_type=jax.ShapeDtypeStruct((NCORE, ACC), jnp.float32),
        scratch_types=[
            pltpu.VMEM((CE, CPC), jnp.int32),
            pltpu.VMEM((CE, CPC), jnp.int32),
            pltpu.VMEM((CE, CPC), jnp.float32),
            pltpu.VMEM((CE, CPC), jnp.float32),
            pltpu.VMEM((WELE,), jnp.int32),
            pltpu.VMEM((WELE,), jnp.int32),
            pltpu.VMEM((WELE,), jnp.float32),
            pltpu.VMEM((WELE,), jnp.float32),
            pltpu.VMEM((ZCHUNK,), jnp.float32),
            pltpu.VMEM_SHARED((ACC,), jnp.float32),
            pltpu.SemaphoreType.DMA,
            pltpu.SemaphoreType.DMA,
            pltpu.SemaphoreType.DMA,
            pltpu.SemaphoreType.DMA,
            pltpu.SemaphoreType.DMA,
            pltpu.SemaphoreType.DMA,
            pltpu.SemaphoreType.DMA,
            pltpu.SemaphoreType.DMA,
            pltpu.SemaphoreType.DMA,
        ],
        compiler_params=pltpu.CompilerParams(use_tc_tiling_on_sc=False),
    )
    parts = scatter(ind2_flat, inter)
    return jnp.concatenate(
        [parts[0].reshape(N_NODES, CPC), parts[1].reshape(N_NODES, CPC)],
        axis=1,
    )


def kernel(ind_2, prop, inter):
    ind2_flat = ind_2.reshape(N_EDGES, 2 * D_FEAT)
    return _run(ind2_flat, inter)


# speedup vs baseline: 150.5712x; 1.0233x over previous
import jax
import jax.numpy as jnp
from jax import lax
from jax.experimental import pallas as pl
from jax.experimental.pallas import tpu as pltpu
from jax.experimental.pallas import tpu_sc as plsc

N_NODES = 10000
N_EDGES = 320000
D_FEAT = 128

NCORE = 2
NSUB = 16
CPC = D_FEAT // NCORE
EPS = N_EDGES // NSUB
CE = 100
NWIN = EPS // CE
NPAIR = NWIN // 2
WELE = CE * CPC
ACC = N_NODES * CPC
ZSLICE = ACC // NSUB
ZCHUNK = 8000
NQ = CPC // 16


def _sc_scatter_kernel(ind2_hbm, inter_hbm, out_hbm,
                       idx0, idx1, val0, val1, fidx0, fidx1, fval0, fval1,
                       zbuf, acc, si0, si1, sv0, sv1, ss0, ss1, ssa0, ssa1,
                       zsem):
    cid = lax.axis_index("c")
    sid = lax.axis_index("s")
    idxb = (idx0, idx1)
    valb = (val0, val1)
    fidxb = (fidx0, fidx1)
    fvalb = (fval0, fval1)
    sidx = (si0, si1)
    sval = (sv0, sv1)
    ssc = (ss0, ss1)
    ssc2 = (ssa0, ssa1)

    zvec = jnp.zeros((16,), jnp.float32)

    def _zero(i, _):
        zbuf[pl.ds(i * 16, 16)] = zvec
        return 0

    lax.fori_loop(0, ZCHUNK // 16, _zero, 0)
    for t in range(ZSLICE // ZCHUNK):
        pltpu.async_copy(zbuf, acc.at[pl.ds(sid * ZSLICE + t * ZCHUNK, ZCHUNK)],
                         zsem)
    for t in range(ZSLICE // ZCHUNK):
        pltpu.make_async_copy(
            zbuf, acc.at[pl.ds(sid * ZSLICE + t * ZCHUNK, ZCHUNK)], zsem
        ).wait()
    plsc.subcore_barrier()

    col0 = lax.iota(jnp.int32, 16)
    cols = [col0 + (16 * q) for q in range(NQ)]
    cbase = cid * CPC

    def _issue_in(w, b):
        e0 = sid * EPS + w * CE
        pltpu.async_copy(ind2_hbm.at[pl.ds(e0, CE), pl.ds(cbase, CPC)],
                         idxb[b], sidx[b])
        pltpu.async_copy(inter_hbm.at[pl.ds(e0, CE), pl.ds(cbase, CPC)],
                         valb[b], sval[b])

    def _wait_in(w, b):
        e0 = sid * EPS + w * CE
        pltpu.make_async_copy(ind2_hbm.at[pl.ds(e0, CE), pl.ds(cbase, CPC)],
                              idxb[b], sidx[b]).wait()
        pltpu.make_async_copy(inter_hbm.at[pl.ds(e0, CE), pl.ds(cbase, CPC)],
                              valb[b], sval[b]).wait()

    H = WELE // 2

    def _wait_scatter(b):
        pltpu.make_async_copy(fvalb[b].at[pl.ds(0, H)],
                              acc.at[fidxb[b].at[pl.ds(0, H)]], ssc[b]).wait()
        pltpu.make_async_copy(fvalb[b].at[pl.ds(H, H)],
                              acc.at[fidxb[b].at[pl.ds(H, H)]], ssc2[b]).wait()

    def _half(k, w, b):
        _wait_in(w, b)

        @pl.when(k > 0)
        def _():
            _wait_scatter(b)

        def _repack(r, _):
            for q in range(NQ):
                src = pl.ds(16 * q, 16)
                dst = pl.ds(r * CPC + 16 * q, 16)
                fidxb[b][dst] = idxb[b][r, src] * CPC + cols[q]
                fvalb[b][dst] = valb[b][r, src]
            return 0

        lax.fori_loop(0, CE, _repack, 0)
        pltpu.async_copy(fvalb[b].at[pl.ds(0, H)],
                         acc.at[fidxb[b].at[pl.ds(0, H)]], ssc[b], add=True)
        pltpu.async_copy(fvalb[b].at[pl.ds(H, H)],
                         acc.at[fidxb[b].at[pl.ds(H, H)]], ssc2[b], add=True)

    def _pair(k, _):
        w0 = 2 * k
        _issue_in(w0 + 1, 1)
        _half(k, w0, 0)

        @pl.when(k + 1 < NPAIR)
        def _():
            _issue_in(w0 + 2, 0)

        _half(k, w0 + 1, 1)
        return 0

    _issue_in(0, 0)
    lax.fori_loop(0, NPAIR, _pair, 0)
    _wait_scatter(0)
    _wait_scatter(1)
    plsc.subcore_barrier()

    rbase = sid * (N_NODES // NSUB)

    def _drain_issue(r, _):
        row = rbase + r
        pltpu.async_copy(acc.at[pl.ds(row * CPC, CPC)],
                         out_hbm.at[row, pl.ds(cbase, CPC)], zsem)
        return 0

    def _drain_wait(r, _):
        row = rbase + r
        pltpu.make_async_copy(acc.at[pl.ds(row * CPC, CPC)],
                              out_hbm.at[row, pl.ds(cbase, CPC)], zsem).wait()
        return 0

    lax.fori_loop(0, N_NODES // NSUB, _drain_issue, 0)
    lax.fori_loop(0, N_NODES // NSUB, _drain_wait, 0)


@jax.jit
def _run(ind2_flat, inter):
    mesh = plsc.VectorSubcoreMesh(core_axis_name="c", subcore_axis_name="s")
    scatter = pl.kernel(
        _sc_scatter_kernel,
        mesh=mesh,
        out_type=jax.ShapeDtypeStruct((N_NODES, D_FEAT), jnp.float32),
        scratch_types=[
            pltpu.VMEM((CE, CPC), jnp.int32),
            pltpu.VMEM((CE, CPC), jnp.int32),
            pltpu.VMEM((CE, CPC), jnp.float32),
            pltpu.VMEM((CE, CPC), jnp.float32),
            pltpu.VMEM((WELE,), jnp.int32),
            pltpu.VMEM((WELE,), jnp.int32),
            pltpu.VMEM((WELE,), jnp.float32),
            pltpu.VMEM((WELE,), jnp.float32),
            pltpu.VMEM((ZCHUNK,), jnp.float32),
            pltpu.VMEM_SHARED((ACC,), jnp.float32),
            pltpu.SemaphoreType.DMA,
            pltpu.SemaphoreType.DMA,
            pltpu.SemaphoreType.DMA,
            pltpu.SemaphoreType.DMA,
            pltpu.SemaphoreType.DMA,
            pltpu.SemaphoreType.DMA,
            pltpu.SemaphoreType.DMA,
            pltpu.SemaphoreType.DMA,
            pltpu.SemaphoreType.DMA,
        ],
        compiler_params=pltpu.CompilerParams(use_tc_tiling_on_sc=False),
    )
    return scatter(ind2_flat, inter)


def kernel(ind_2, prop, inter):
    ind2_flat = ind_2.reshape(N_EDGES, 2 * D_FEAT)
    return _run(ind2_flat, inter)


# speedup vs baseline: 151.3547x; 1.0052x over previous
import jax
import jax.numpy as jnp
from jax import lax
from jax.experimental import pallas as pl
from jax.experimental.pallas import tpu as pltpu
from jax.experimental.pallas import tpu_sc as plsc

N_NODES = 10000
N_EDGES = 320000
D_FEAT = 128

NCORE = 2
NSUB = 16
CPC = D_FEAT // NCORE
EPS = N_EDGES // NSUB
CE = 100
NWIN = EPS // CE
NPAIR = NWIN // 2
WELE = CE * CPC
ACC = N_NODES * CPC
ZSLICE = ACC // NSUB
ZCHUNK = 8000
NQ = CPC // 16


def _sc_scatter_kernel(ind2_hbm, inter_hbm, out_hbm,
                       idx0, idx1, val0, val1, fidx0, fidx1, fval0, fval1,
                       zbuf, acc, si0, si1, sv0, sv1, ss0, ss1, ssa0, ssa1,
                       zsem):
    cid = lax.axis_index("c")
    sid = lax.axis_index("s")
    cbase = cid * CPC

    def _issue_in0():
        e0 = sid * EPS
        pltpu.async_copy(ind2_hbm.at[pl.ds(e0, CE), pl.ds(cbase, CPC)],
                         idx0, si0)
        pltpu.async_copy(inter_hbm.at[pl.ds(e0, CE), pl.ds(cbase, CPC)],
                         val0, sv0)
    idxb = (idx0, idx1)
    valb = (val0, val1)
    fidxb = (fidx0, fidx1)
    fvalb = (fval0, fval1)
    sidx = (si0, si1)
    sval = (sv0, sv1)
    ssc = (ss0, ss1)
    ssc2 = (ssa0, ssa1)

    zvec = jnp.zeros((16,), jnp.float32)

    def _zero(i, _):
        zbuf[pl.ds(i * 16, 16)] = zvec
        return 0

    _issue_in0()
    lax.fori_loop(0, ZCHUNK // 16, _zero, 0)
    for t in range(ZSLICE // ZCHUNK):
        pltpu.async_copy(zbuf, acc.at[pl.ds(sid * ZSLICE + t * ZCHUNK, ZCHUNK)],
                         zsem)
    for t in range(ZSLICE // ZCHUNK):
        pltpu.make_async_copy(
            zbuf, acc.at[pl.ds(sid * ZSLICE + t * ZCHUNK, ZCHUNK)], zsem
        ).wait()
    plsc.subcore_barrier()

    col0 = lax.iota(jnp.int32, 16)
    cols = [col0 + (16 * q) for q in range(NQ)]

    def _issue_in(w, b):
        e0 = sid * EPS + w * CE
        pltpu.async_copy(ind2_hbm.at[pl.ds(e0, CE), pl.ds(cbase, CPC)],
                         idxb[b], sidx[b])
        pltpu.async_copy(inter_hbm.at[pl.ds(e0, CE), pl.ds(cbase, CPC)],
                         valb[b], sval[b])

    def _wait_in(w, b):
        e0 = sid * EPS + w * CE
        pltpu.make_async_copy(ind2_hbm.at[pl.ds(e0, CE), pl.ds(cbase, CPC)],
                              idxb[b], sidx[b]).wait()
        pltpu.make_async_copy(inter_hbm.at[pl.ds(e0, CE), pl.ds(cbase, CPC)],
                              valb[b], sval[b]).wait()

    H = WELE // 2

    def _wait_scatter(b):
        pltpu.make_async_copy(fvalb[b].at[pl.ds(0, H)],
                              acc.at[fidxb[b].at[pl.ds(0, H)]], ssc[b]).wait()
        pltpu.make_async_copy(fvalb[b].at[pl.ds(H, H)],
                              acc.at[fidxb[b].at[pl.ds(H, H)]], ssc2[b]).wait()

    def _half(k, w, b):
        _wait_in(w, b)

        @pl.when(k > 0)
        def _():
            _wait_scatter(b)

        def _repack(r, _):
            for q in range(NQ):
                src = pl.ds(16 * q, 16)
                dst = pl.ds(r * CPC + 16 * q, 16)
                fidxb[b][dst] = idxb[b][r, src] * CPC + cols[q]
                fvalb[b][dst] = valb[b][r, src]
            return 0

        lax.fori_loop(0, CE, _repack, 0)
        pltpu.async_copy(fvalb[b].at[pl.ds(0, H)],
                         acc.at[fidxb[b].at[pl.ds(0, H)]], ssc[b], add=True)
        pltpu.async_copy(fvalb[b].at[pl.ds(H, H)],
                         acc.at[fidxb[b].at[pl.ds(H, H)]], ssc2[b], add=True)

    def _pair(k, _):
        w0 = 2 * k
        _issue_in(w0 + 1, 1)
        _half(k, w0, 0)

        @pl.when(k + 1 < NPAIR)
        def _():
            _issue_in(w0 + 2, 0)

        _half(k, w0 + 1, 1)
        return 0

    lax.fori_loop(0, NPAIR, _pair, 0)
    _wait_scatter(0)
    _wait_scatter(1)
    plsc.subcore_barrier()

    rbase = sid * (N_NODES // NSUB)

    def _drain_issue(g, _):
        for u in range(5):
            row = rbase + g * 5 + u
            pltpu.async_copy(acc.at[pl.ds(row * CPC, CPC)],
                             out_hbm.at[row, pl.ds(cbase, CPC)], zsem)
        return 0

    def _drain_wait(g, _):
        for u in range(5):
            row = rbase + g * 5 + u
            pltpu.make_async_copy(acc.at[pl.ds(row * CPC, CPC)],
                                  out_hbm.at[row, pl.ds(cbase, CPC)],
                                  zsem).wait()
        return 0

    lax.fori_loop(0, N_NODES // NSUB // 5, _drain_issue, 0)
    lax.fori_loop(0, N_NODES // NSUB // 5, _drain_wait, 0)


@jax.jit
def _run(ind2_flat, inter):
    mesh = plsc.VectorSubcoreMesh(core_axis_name="c", subcore_axis_name="s")
    scatter = pl.kernel(
        _sc_scatter_kernel,
        mesh=mesh,
        out_type=jax.ShapeDtypeStruct((N_NODES, D_FEAT), jnp.float32),
        scratch_types=[
            pltpu.VMEM((CE, CPC), jnp.int32),
            pltpu.VMEM((CE, CPC), jnp.int32),
            pltpu.VMEM((CE, CPC), jnp.float32),
            pltpu.VMEM((CE, CPC), jnp.float32),
            pltpu.VMEM((WELE,), jnp.int32),
            pltpu.VMEM((WELE,), jnp.int32),
            pltpu.VMEM((WELE,), jnp.float32),
            pltpu.VMEM((WELE,), jnp.float32),
            pltpu.VMEM((ZCHUNK,), jnp.float32),
            pltpu.VMEM_SHARED((ACC,), jnp.float32),
            pltpu.SemaphoreType.DMA,
            pltpu.SemaphoreType.DMA,
            pltpu.SemaphoreType.DMA,
            pltpu.SemaphoreType.DMA,
            pltpu.SemaphoreType.DMA,
            pltpu.SemaphoreType.DMA,
            pltpu.SemaphoreType.DMA,
            pltpu.SemaphoreType.DMA,
            pltpu.SemaphoreType.DMA,
        ],
        compiler_params=pltpu.CompilerParams(use_tc_tiling_on_sc=False),
    )
    return scatter(ind2_flat, inter)


def kernel(ind_2, prop, inter):
    ind2_flat = ind_2.reshape(N_EDGES, 2 * D_FEAT)
    return _run(ind2_flat, inter)
